# Initial kernel scaffold; baseline (speedup 1.0000x reference)
import jax
import jax.numpy as jnp
from jax.experimental import pallas as pl


def kernel(x, token_table, pos_table):
    raise NotImplementedError("write your pallas kernel here")


# speedup vs baseline: 1.0972x; 1.0972x over previous
import functools

import jax
import jax.numpy as jnp
from jax import lax
from jax.experimental import pallas as pl
from jax.experimental.pallas import tpu as pltpu
from jax.experimental.pallas import tpu_sc as plsc

_NC = 2
_NS = 16
_NW = _NC * _NS
_LANES = 16


def _build_kernel(B, S, V, D):
    assert S % _NW == 0
    P = S // _NW
    assert D % _LANES == 0
    mesh = plsc.VectorSubcoreMesh(
        core_axis_name="c", subcore_axis_name="s",
        num_cores=_NC, num_subcores=_NS,
    )

    @functools.partial(
        pl.kernel,
        out_type=jax.ShapeDtypeStruct((B * S, D), jnp.float32),
        mesh=mesh,
        scratch_types=[
            pltpu.VMEM((P,), jnp.int32),
            pltpu.VMEM((P,), jnp.int32),
            pltpu.VMEM((P, D), jnp.float32),
            pltpu.VMEM((P, D), jnp.float32),
            pltpu.VMEM((P, D), jnp.float32),
            pltpu.SemaphoreType.DMA,
            pltpu.SemaphoreType.DMA,
        ],
    )
    def k(x_hbm, tok_hbm, pos_hbm, out_hbm,
          idx0, idx1, rows0, rows1, pos_v, sem0, sem1):
        wid = lax.axis_index("s") * _NC + lax.axis_index("c")
        p0 = wid * P

        pltpu.sync_copy(pos_hbm.at[pl.ds(p0, P), :], pos_v)

        bufs = ((idx0, rows0, sem0), (idx1, rows1, sem1))

        def start_gather(b, buf):
            idx_v, rows_v, sem = bufs[buf]
            pltpu.sync_copy(x_hbm.at[pl.ds(b * S + p0, P)], idx_v)
            return pltpu.async_copy(tok_hbm.at[idx_v], rows_v, sem)

        def add_pos(rows_v):
            def body(i, carry):
                for j in range(D // _LANES):
                    sl = pl.ds(j * _LANES, _LANES)
                    plsc.addupdate(rows_v.at[i, sl], pos_v[i, sl])
                return carry
            lax.fori_loop(0, P, body, 0, unroll=2)

        desc = start_gather(0, 0)
        for b in range(B):
            buf = b % 2
            _, rows_v, _ = bufs[buf]
            nxt_desc = start_gather(b + 1, 1 - buf) if b + 1 < B else None
            desc.wait()
            add_pos(rows_v)
            pltpu.sync_copy(rows_v, out_hbm.at[pl.ds(b * S + p0, P), :])
            desc = nxt_desc

    return k


def kernel(x, token_table, pos_table):
    B, S = x.shape
    V, D = token_table.shape
    k = _build_kernel(B, S, V, D)
    out = k(x.reshape(-1).astype(jnp.int32), token_table, pos_table)
    return out.reshape(B, S, D)
